# Initial kernel scaffold; baseline (speedup 1.0000x reference)
import jax
import jax.numpy as jnp
from jax.experimental import pallas as pl


def kernel(x, edge_index, W1, al1, ar1, b1, W2, al2, ar2, b2, Wn1, bn1, Wn2, bn2, Wg1, bg1, Wg2, bg2):
    raise NotImplementedError("write your pallas kernel here")


# speedup vs baseline: 13.6869x; 13.6869x over previous
import dataclasses
import functools

import jax
import jax.numpy as jnp
from jax import lax
from jax.experimental import pallas as pl
from jax.experimental.pallas import tpu as pltpu
from jax.experimental.pallas import tpu_sc as plsc

_N = 10000
_E = 320000
_F_IN = 128
_HID = 64
_HEADS = 4

_R = 400
_GRID = _N // _R
_CH = 80
_NCHUNKS = _E // _CH
_NSUB = 16
_NCORE = 2
_NPAD = 10240
_ROWS_PER_SUB = _NPAD // _NSUB

_mesh = plsc.VectorSubcoreMesh(core_axis_name="c", subcore_axis_name="s",
                               num_cores=_NCORE, num_subcores=_NSUB)

_SC_PARAMS = pltpu.CompilerParams()
if "needs_layout_passes" in getattr(pltpu.CompilerParams, "__dataclass_fields__", {}):
    _SC_PARAMS = dataclasses.replace(_SC_PARAMS, needs_layout_passes=False)


def _leaky(x):
    return jnp.where(x >= 0, x, 0.2 * x)


def _proj1_body(x_ref, w_ref, al_ref, ar_ref,
                h1a_ref, h1b_ref, el_ref, er_ref, mel_ref, mer_ref):
    i = pl.program_id(0)
    h = lax.dot_general(x_ref[...], w_ref[...], (((1,), (1,)), ((), ())),
                        precision=lax.Precision.HIGHEST,
                        preferred_element_type=jnp.float32)
    els, ers = [], []
    for hh in range(_HEADS):
        blk = h[:, hh * _HID:(hh + 1) * _HID]
        els.append(jnp.dot(blk, al_ref[hh, :], precision=lax.Precision.HIGHEST))
        ers.append(jnp.dot(blk, ar_ref[hh, :], precision=lax.Precision.HIGHEST))
    el = jnp.stack(els, axis=1)
    er = jnp.stack(ers, axis=1)
    el_ref[...] = el
    er_ref[...] = er
    h1a_ref[...] = h[:, :128]
    h1b_ref[...] = h[:, 128:]

    @pl.when(i == 0)
    def _():
        mel_ref[...] = jnp.full((_HEADS, 16), -jnp.inf, jnp.float32)
        mer_ref[...] = jnp.full((_HEADS, 16), -jnp.inf, jnp.float32)

    mel_ref[...] = jnp.maximum(mel_ref[...], jnp.max(el, axis=0)[:, None])
    mer_ref[...] = jnp.maximum(mer_ref[...], jnp.max(er, axis=0)[:, None])


def _proj1(x, W1, al1, ar1):
    return pl.pallas_call(
        _proj1_body,
        grid=(_GRID,),
        in_specs=[
            pl.BlockSpec((_R, _F_IN), lambda i: (i, 0)),
            pl.BlockSpec((_HEADS * _HID, _F_IN), lambda i: (0, 0)),
            pl.BlockSpec((_HEADS, _HID), lambda i: (0, 0)),
            pl.BlockSpec((_HEADS, _HID), lambda i: (0, 0)),
        ],
        out_specs=[
            pl.BlockSpec((_R, 128), lambda i: (i, 0)),
            pl.BlockSpec((_R, 128), lambda i: (i, 0)),
            pl.BlockSpec((_R, _HEADS), lambda i: (i, 0)),
            pl.BlockSpec((_R, _HEADS), lambda i: (i, 0)),
            pl.BlockSpec((_HEADS, 16), lambda i: (0, 0)),
            pl.BlockSpec((_HEADS, 16), lambda i: (0, 0)),
        ],
        out_shape=[
            jax.ShapeDtypeStruct((_N, 128), jnp.float32),
            jax.ShapeDtypeStruct((_N, 128), jnp.float32),
            jax.ShapeDtypeStruct((_N, _HEADS), jnp.float32),
            jax.ShapeDtypeStruct((_N, _HEADS), jnp.float32),
            jax.ShapeDtypeStruct((_HEADS, 16), jnp.float32),
            jax.ShapeDtypeStruct((_HEADS, 16), jnp.float32),
        ],
    )(x, W1, al1, ar1)


def _edge1_body(h1a_hbm, h1b_hbm, src_hbm, dst_hbm, elt_hbm, ert_hbm, mel_hbm, mer_hbm, zer_hbm,
                out_hbm,
                outsh, elv, erv, srcv, dstv, featv, stg, wbuf, cbuf, sem):
    c = lax.axis_index("c")
    s = lax.axis_index("s")
    lane = lax.iota(jnp.int32, 16)
    zero16 = jnp.zeros((16,), jnp.float32)

    @pl.loop(0, _CH)
    def _zs(r):
        for k in range(3):
            stg[r, pl.ds(80 + k * 16, 16)] = zero16

    for ph in range(2):
        head = 2 * ph + c

        pltpu.sync_copy(zer_hbm,
                        outsh.at[pl.ds(s * _ROWS_PER_SUB, _ROWS_PER_SUB), :])
        pltpu.sync_copy(elt_hbm.at[pl.ds(head * _N, _N)], elv)
        pltpu.sync_copy(ert_hbm.at[pl.ds(head * _N, _N)], erv)
        pltpu.sync_copy(mel_hbm.at[pl.ds(head * 16, 16)], cbuf)
        a = cbuf[...]
        pltpu.sync_copy(mer_hbm.at[pl.ds(head * 16, 16)], cbuf)
        c0 = _leaky(a + cbuf[...])

        plsc.subcore_barrier()

        tab = h1a_hbm if ph == 0 else h1b_hbm

        @pl.loop(0, _NCHUNKS // _NSUB)
        def _chunk(j):
            off = (s + _NSUB * j) * _CH
            pltpu.sync_copy(src_hbm.at[pl.ds(off, _CH)], srcv)
            pltpu.sync_copy(dst_hbm.at[pl.ds(off, _CH)], dstv)
            gat = pltpu.make_async_copy(tab.at[srcv], featv, sem)
            gat.start()

            @pl.loop(0, _CH // 16)
            def _grp(g):
                src16 = srcv[pl.ds(g * 16, 16)]
                dst16 = dstv[pl.ds(g * 16, 16)]
                e0 = _leaky(plsc.load_gather(elv, [src16]) +
                            plsc.load_gather(erv, [dst16]))
                wbuf[pl.ds(g * 16, 16)] = jnp.exp(e0 - c0)

            gat.wait()

            @pl.loop(0, _CH // 16)
            def _grp2(g):
                for t in range(16):
                    row = g * 16 + t
                    wb = plsc.load_gather(
                        wbuf, [jnp.full((16,), 0, jnp.int32) + row])
                    for k in range(4):
                        v = featv[row, pl.ds(c * 64 + k * 16, 16)]
                        stg[row, pl.ds(k * 16, 16)] = v * wb
                    stg[row, pl.ds(64, 16)] = jnp.where(lane == 0, wb,
                                                        zero16)

            pltpu.sync_copy(stg, outsh.at[dstv], add=True)

        plsc.subcore_barrier()
        pltpu.sync_copy(outsh.at[pl.ds(s * _ROWS_PER_SUB, _ROWS_PER_SUB), :],
                        out_hbm.at[ph, c,
                                   pl.ds(s * _ROWS_PER_SUB, _ROWS_PER_SUB), :])


def _edge_pass1(h1a, h1b, src, dst, elt, ert, mel, mer):
    zer = jnp.zeros((_ROWS_PER_SUB, 128), jnp.float32)
    k = pl.kernel(
        _edge1_body,
        out_type=jax.ShapeDtypeStruct((2, 2, _NPAD, 128), jnp.float32),
        mesh=_mesh,
        compiler_params=_SC_PARAMS,
        scratch_types=[
            pltpu.VMEM_SHARED((_NPAD, 128), jnp.float32),
            pltpu.VMEM((_N,), jnp.float32),
            pltpu.VMEM((_N,), jnp.float32),
            pltpu.VMEM((_CH,), jnp.int32),
            pltpu.VMEM((_CH,), jnp.int32),
            pltpu.VMEM((_CH, 128), jnp.float32),
            pltpu.VMEM((_CH, 128), jnp.float32),
            pltpu.VMEM((_CH,), jnp.float32),
            pltpu.VMEM((16,), jnp.float32),
            pltpu.SemaphoreType.DMA,
        ],
    )
    return k(h1a, h1b, src, dst, elt, ert, mel, mer, zer)


def _norm1_body(sc_ref, b1_ref, w2_ref, al2_ref, ar2_ref,
                h2_ref, el2_ref, er2_ref, mel_ref, mer_ref):
    i = pl.program_id(0)
    cols = []
    for ph in range(2):
        for cc in range(2):
            plane = sc_ref[ph, cc]
            d = jnp.maximum(plane[:, 64:65], 1e-9)
            cols.append(plane[:, 0:64] / d)
    h1f = jnp.concatenate(cols, axis=1) + b1_ref[...]
    h2 = lax.dot_general(h1f, w2_ref[...], (((1,), (1,)), ((), ())),
                         precision=lax.Precision.HIGHEST,
                         preferred_element_type=jnp.float32)
    h2_ref[...] = jnp.concatenate(
        [h2, jnp.zeros((h2.shape[0], 64), jnp.float32)], axis=1)
    el2 = jnp.dot(h2, al2_ref[0, :], precision=lax.Precision.HIGHEST)
    er2 = jnp.dot(h2, ar2_ref[0, :], precision=lax.Precision.HIGHEST)
    el2_ref[...] = el2[:, None]
    er2_ref[...] = er2[:, None]

    @pl.when(i == 0)
    def _():
        mel_ref[...] = jnp.full((1, 16), -jnp.inf, jnp.float32)
        mer_ref[...] = jnp.full((1, 16), -jnp.inf, jnp.float32)

    mel_ref[...] = jnp.maximum(mel_ref[...], jnp.max(el2))
    mer_ref[...] = jnp.maximum(mer_ref[...], jnp.max(er2))


def _norm1_proj2(sc1, b1, W2, al2, ar2):
    return pl.pallas_call(
        _norm1_body,
        grid=(_GRID,),
        in_specs=[
            pl.BlockSpec((2, 2, _R, 128), lambda i: (0, 0, i, 0)),
            pl.BlockSpec((1, _HEADS * _HID), lambda i: (0, 0)),
            pl.BlockSpec((_HID, _HEADS * _HID), lambda i: (0, 0)),
            pl.BlockSpec((1, _HID), lambda i: (0, 0)),
            pl.BlockSpec((1, _HID), lambda i: (0, 0)),
        ],
        out_specs=[
            pl.BlockSpec((_R, 128), lambda i: (i, 0)),
            pl.BlockSpec((_R, 1), lambda i: (i, 0)),
            pl.BlockSpec((_R, 1), lambda i: (i, 0)),
            pl.BlockSpec((1, 16), lambda i: (0, 0)),
            pl.BlockSpec((1, 16), lambda i: (0, 0)),
        ],
        out_shape=[
            jax.ShapeDtypeStruct((_N, 128), jnp.float32),
            jax.ShapeDtypeStruct((_N, 1), jnp.float32),
            jax.ShapeDtypeStruct((_N, 1), jnp.float32),
            jax.ShapeDtypeStruct((1, 16), jnp.float32),
            jax.ShapeDtypeStruct((1, 16), jnp.float32),
        ],
    )(sc1, b1, W2, al2, ar2)


def _edge2_body(h2_hbm, src_hbm, dst_hbm, el_hbm, er_hbm, mel_hbm, mer_hbm, zer_hbm,
                out_hbm,
                outsh, elv, erv, srcv, dstv, featv, stg, wbuf, cbuf, sem):
    c = lax.axis_index("c")
    s = lax.axis_index("s")
    lane = lax.iota(jnp.int32, 16)
    zero16 = jnp.zeros((16,), jnp.float32)

    pltpu.sync_copy(zer_hbm, outsh.at[pl.ds(s * _ROWS_PER_SUB, _ROWS_PER_SUB), :])
    pltpu.sync_copy(el_hbm, elv)
    pltpu.sync_copy(er_hbm, erv)
    pltpu.sync_copy(mel_hbm, cbuf)
    a = cbuf[...]
    pltpu.sync_copy(mer_hbm, cbuf)
    c0 = _leaky(a + cbuf[...])

    @pl.loop(0, _CH)
    def _zs(r):
        for k in range(3):
            stg[r, pl.ds(80 + k * 16, 16)] = zero16

    plsc.subcore_barrier()

    half = _NCHUNKS // 2

    @pl.loop(0, half // _NSUB)
    def _chunk(j):
        off = (c * half + s + _NSUB * j) * _CH
        pltpu.sync_copy(src_hbm.at[pl.ds(off, _CH)], srcv)
        pltpu.sync_copy(dst_hbm.at[pl.ds(off, _CH)], dstv)
        gat = pltpu.make_async_copy(h2_hbm.at[srcv], featv, sem)
        gat.start()

        @pl.loop(0, _CH // 16)
        def _grp(g):
            src16 = srcv[pl.ds(g * 16, 16)]
            dst16 = dstv[pl.ds(g * 16, 16)]
            e0 = _leaky(plsc.load_gather(elv, [src16]) +
                        plsc.load_gather(erv, [dst16]))
            wbuf[pl.ds(g * 16, 16)] = jnp.exp(e0 - c0)

        gat.wait()

        @pl.loop(0, _CH // 16)
        def _grp2(g):
            for t in range(16):
                row = g * 16 + t
                wb = plsc.load_gather(wbuf, [jnp.full((16,), 0, jnp.int32) + row])
                for k in range(4):
                    v = featv[row, pl.ds(k * 16, 16)]
                    stg[row, pl.ds(k * 16, 16)] = v * wb
                stg[row, pl.ds(64, 16)] = jnp.where(lane == 0, wb, zero16)

        pltpu.sync_copy(stg, outsh.at[dstv], add=True)

    plsc.subcore_barrier()
    pltpu.sync_copy(outsh.at[pl.ds(s * _ROWS_PER_SUB, _ROWS_PER_SUB), :],
                    out_hbm.at[c, pl.ds(s * _ROWS_PER_SUB, _ROWS_PER_SUB), :])


def _edge_pass2(h2, src, dst, el2, er2, mel, mer):
    zer = jnp.zeros((_ROWS_PER_SUB, 128), jnp.float32)
    k = pl.kernel(
        _edge2_body,
        out_type=jax.ShapeDtypeStruct((2, _NPAD, 128), jnp.float32),
        mesh=_mesh,
        compiler_params=_SC_PARAMS,
        scratch_types=[
            pltpu.VMEM_SHARED((_NPAD, 128), jnp.float32),
            pltpu.VMEM((_N,), jnp.float32),
            pltpu.VMEM((_N,), jnp.float32),
            pltpu.VMEM((_CH,), jnp.int32),
            pltpu.VMEM((_CH,), jnp.int32),
            pltpu.VMEM((_CH, 128), jnp.float32),
            pltpu.VMEM((_CH, 128), jnp.float32),
            pltpu.VMEM((_CH,), jnp.float32),
            pltpu.VMEM((16,), jnp.float32),
            pltpu.SemaphoreType.DMA,
        ],
    )
    return k(h2, src, dst, el2, er2, mel, mer, zer)


def _final_body(p_ref, b2_ref, wn1_ref, bn1_ref, wn2_ref, bn2_ref,
                wg1_ref, bg1_ref, wg2_ref, bg2_ref,
                nl_ref, gl_ref, hsum_ref):
    i = pl.program_id(0)
    f = p_ref[0, :, 0:64] + p_ref[1, :, 0:64]
    d = jnp.maximum(p_ref[0, :, 64:65] + p_ref[1, :, 64:65], 1e-9)
    h = f / d + b2_ref[...]
    t = jnp.maximum(
        lax.dot_general(h, wn1_ref[...], (((1,), (1,)), ((), ())),
                        precision=lax.Precision.HIGHEST,
                        preferred_element_type=jnp.float32) + bn1_ref[...], 0.0)
    nl_ref[...] = lax.dot_general(t, wn2_ref[...], (((1,), (1,)), ((), ())),
                                  precision=lax.Precision.HIGHEST,
                                  preferred_element_type=jnp.float32) + bn2_ref[...]

    @pl.when(i == 0)
    def _():
        hsum_ref[...] = jnp.zeros((1, _HID), jnp.float32)

    hsum_ref[...] += jnp.sum(h, axis=0, keepdims=True)

    @pl.when(i == _GRID - 1)
    def _():
        hg = hsum_ref[...] / float(_N)
        tg = jnp.maximum(
            lax.dot_general(hg, wg1_ref[...], (((1,), (1,)), ((), ())),
                            precision=lax.Precision.HIGHEST,
                            preferred_element_type=jnp.float32) + bg1_ref[...], 0.0)
        gl_ref[...] = lax.dot_general(tg, wg2_ref[...], (((1,), (1,)), ((), ())),
                                      precision=lax.Precision.HIGHEST,
                                      preferred_element_type=jnp.float32) + bg2_ref[...]


def _final(p2, b2, Wn1, bn1, Wn2, bn2, Wg1, bg1, Wg2, bg2):
    return pl.pallas_call(
        _final_body,
        grid=(_GRID,),
        in_specs=[
            pl.BlockSpec((2, _R, 128), lambda i: (0, i, 0)),
            pl.BlockSpec((1, _HID), lambda i: (0, 0)),
            pl.BlockSpec((_HID, _HID), lambda i: (0, 0)),
            pl.BlockSpec((1, _HID), lambda i: (0, 0)),
            pl.BlockSpec((2, _HID), lambda i: (0, 0)),
            pl.BlockSpec((1, 2), lambda i: (0, 0)),
            pl.BlockSpec((_HID, _HID), lambda i: (0, 0)),
            pl.BlockSpec((1, _HID), lambda i: (0, 0)),
            pl.BlockSpec((2, _HID), lambda i: (0, 0)),
            pl.BlockSpec((1, 2), lambda i: (0, 0)),
        ],
        out_specs=[
            pl.BlockSpec((_R, 2), lambda i: (i, 0)),
            pl.BlockSpec((1, 2), lambda i: (0, 0)),
            pl.BlockSpec((1, _HID), lambda i: (0, 0)),
        ],
        out_shape=[
            jax.ShapeDtypeStruct((_N, 2), jnp.float32),
            jax.ShapeDtypeStruct((1, 2), jnp.float32),
            jax.ShapeDtypeStruct((1, _HID), jnp.float32),
        ],
    )(p2, b2, Wn1, bn1, Wn2, bn2, Wg1, bg1, Wg2, bg2)


def kernel(x, edge_index, W1, al1, ar1, b1, W2, al2, ar2, b2,
           Wn1, bn1, Wn2, bn2, Wg1, bg1, Wg2, bg2):
    h1a, h1b, el1, er1, mel1, mer1 = _proj1(x, W1, al1, ar1)
    src = edge_index[0]
    dst = edge_index[1]
    sc1 = _edge_pass1(h1a, h1b, src, dst, el1.T.reshape(-1), er1.T.reshape(-1),
                      mel1.reshape(-1), mer1.reshape(-1))
    h2, el2, er2, mel2, mer2 = _norm1_proj2(sc1, b1.reshape(1, -1),
                                            W2, al2, ar2)
    sc2 = _edge_pass2(h2, src, dst, el2.reshape(_N), er2.reshape(_N),
                      mel2.reshape(-1), mer2.reshape(-1))
    node_logits, graph_logits, _ = _final(sc2, b2.reshape(1, -1),
                                          Wn1, bn1.reshape(1, -1),
                                          Wn2, bn2.reshape(1, -1),
                                          Wg1, bg1.reshape(1, -1),
                                          Wg2, bg2.reshape(1, -1))
    return node_logits, graph_logits


# speedup vs baseline: 13.8030x; 1.0085x over previous
import dataclasses

import jax
import jax.numpy as jnp
from jax import lax
from jax.experimental import pallas as pl
from jax.experimental.pallas import tpu as pltpu
from jax.experimental.pallas import tpu_sc as plsc

_N = 10000
_E = 320000
_F_IN = 128
_HID = 64
_HEADS = 4

_R = 400
_GRID = _N // _R
_CH = 80
_NCHUNKS = _E // _CH
_NSUB = 16
_NCORE = 2
_NPAD = 10240
_ROWS_PER_SUB = _NPAD // _NSUB

_mesh = plsc.VectorSubcoreMesh(core_axis_name="c", subcore_axis_name="s",
                               num_cores=_NCORE, num_subcores=_NSUB)

_SC_PARAMS = pltpu.CompilerParams()
if "needs_layout_passes" in getattr(pltpu.CompilerParams, "__dataclass_fields__", {}):
    _SC_PARAMS = dataclasses.replace(_SC_PARAMS, needs_layout_passes=False)


def _leaky(x):
    return jnp.where(x >= 0, x, 0.2 * x)


def _proj1_body(x_ref, w_ref, al_ref, ar_ref,
                h1a_ref, h1b_ref, el_ref, er_ref, mel_ref, mer_ref):
    i = pl.program_id(0)
    h = lax.dot_general(x_ref[...], w_ref[...], (((1,), (1,)), ((), ())),
                        preferred_element_type=jnp.float32)
    els, ers = [], []
    for hh in range(_HEADS):
        blk = h[:, hh * _HID:(hh + 1) * _HID]
        els.append(jnp.sum(blk * al_ref[hh, :][None, :], axis=1))
        ers.append(jnp.sum(blk * ar_ref[hh, :][None, :], axis=1))
    el = jnp.stack(els, axis=1)
    er = jnp.stack(ers, axis=1)
    el_ref[...] = el
    er_ref[...] = er
    h1a_ref[...] = h[:, :128]
    h1b_ref[...] = h[:, 128:]

    @pl.when(i == 0)
    def _():
        mel_ref[...] = jnp.full((_HEADS, 16), -jnp.inf, jnp.float32)
        mer_ref[...] = jnp.full((_HEADS, 16), -jnp.inf, jnp.float32)

    mel_ref[...] = jnp.maximum(mel_ref[...], jnp.max(el, axis=0)[:, None])
    mer_ref[...] = jnp.maximum(mer_ref[...], jnp.max(er, axis=0)[:, None])


def _proj1(x, W1, al1, ar1):
    return pl.pallas_call(
        _proj1_body,
        grid=(_GRID,),
        in_specs=[
            pl.BlockSpec((_R, _F_IN), lambda i: (i, 0)),
            pl.BlockSpec((_HEADS * _HID, _F_IN), lambda i: (0, 0)),
            pl.BlockSpec((_HEADS, _HID), lambda i: (0, 0)),
            pl.BlockSpec((_HEADS, _HID), lambda i: (0, 0)),
        ],
        out_specs=[
            pl.BlockSpec((_R, 128), lambda i: (i, 0)),
            pl.BlockSpec((_R, 128), lambda i: (i, 0)),
            pl.BlockSpec((_R, _HEADS), lambda i: (i, 0)),
            pl.BlockSpec((_R, _HEADS), lambda i: (i, 0)),
            pl.BlockSpec((_HEADS, 16), lambda i: (0, 0)),
            pl.BlockSpec((_HEADS, 16), lambda i: (0, 0)),
        ],
        out_shape=[
            jax.ShapeDtypeStruct((_N, 128), jnp.float32),
            jax.ShapeDtypeStruct((_N, 128), jnp.float32),
            jax.ShapeDtypeStruct((_N, _HEADS), jnp.float32),
            jax.ShapeDtypeStruct((_N, _HEADS), jnp.float32),
            jax.ShapeDtypeStruct((_HEADS, 16), jnp.float32),
            jax.ShapeDtypeStruct((_HEADS, 16), jnp.float32),
        ],
    )(x, W1, al1, ar1)


def _edge1_body(h1a_hbm, h1b_hbm, src_hbm, dst_hbm, elt_hbm, ert_hbm, mel_hbm, mer_hbm, zer_hbm,
                out_hbm,
                outsh, elv, erv, srcv, dstv, featv, stg, wbuf, cbuf, sem):
    c = lax.axis_index("c")
    s = lax.axis_index("s")
    lane = lax.iota(jnp.int32, 16)
    zero16 = jnp.zeros((16,), jnp.float32)

    @pl.loop(0, _CH)
    def _zs(r):
        for k in range(3):
            stg[r, pl.ds(80 + k * 16, 16)] = zero16

    for ph in range(2):
        head = 2 * ph + c

        pltpu.sync_copy(zer_hbm,
                        outsh.at[pl.ds(s * _ROWS_PER_SUB, _ROWS_PER_SUB), :])
        pltpu.sync_copy(elt_hbm.at[pl.ds(head * _N, _N)], elv)
        pltpu.sync_copy(ert_hbm.at[pl.ds(head * _N, _N)], erv)
        pltpu.sync_copy(mel_hbm.at[pl.ds(head * 16, 16)], cbuf)
        a = cbuf[...]
        pltpu.sync_copy(mer_hbm.at[pl.ds(head * 16, 16)], cbuf)
        c0 = _leaky(a + cbuf[...])

        plsc.subcore_barrier()

        tab = h1a_hbm if ph == 0 else h1b_hbm

        @pl.loop(0, _NCHUNKS // _NSUB)
        def _chunk(j):
            off = (s + _NSUB * j) * _CH
            pltpu.sync_copy(src_hbm.at[pl.ds(off, _CH)], srcv)
            pltpu.sync_copy(dst_hbm.at[pl.ds(off, _CH)], dstv)
            gat = pltpu.make_async_copy(tab.at[srcv], featv, sem)
            gat.start()

            @pl.loop(0, _CH // 16)
            def _grp(g):
                src16 = srcv[pl.ds(g * 16, 16)]
                dst16 = dstv[pl.ds(g * 16, 16)]
                e0 = _leaky(plsc.load_gather(elv, [src16]) +
                            plsc.load_gather(erv, [dst16]))
                wbuf[pl.ds(g * 16, 16)] = jnp.exp(e0 - c0)

            gat.wait()

            @pl.loop(0, _CH // 16)
            def _grp2(g):
                for t in range(16):
                    row = g * 16 + t
                    wb = plsc.load_gather(
                        wbuf, [jnp.full((16,), 0, jnp.int32) + row])
                    for k in range(4):
                        v = featv[row, pl.ds(c * 64 + k * 16, 16)]
                        stg[row, pl.ds(k * 16, 16)] = v * wb
                    stg[row, pl.ds(64, 16)] = jnp.where(lane == 0, wb,
                                                        zero16)

            pltpu.sync_copy(stg, outsh.at[dstv], add=True)

        plsc.subcore_barrier()
        pltpu.sync_copy(outsh.at[pl.ds(s * _ROWS_PER_SUB, _ROWS_PER_SUB), :],
                        out_hbm.at[ph, c,
                                   pl.ds(s * _ROWS_PER_SUB, _ROWS_PER_SUB), :])


def _edge_pass1(h1a, h1b, src, dst, elt, ert, mel, mer):
    zer = jnp.zeros((_ROWS_PER_SUB, 128), jnp.float32)
    k = pl.kernel(
        _edge1_body,
        out_type=jax.ShapeDtypeStruct((2, 2, _NPAD, 128), jnp.float32),
        mesh=_mesh,
        compiler_params=_SC_PARAMS,
        scratch_types=[
            pltpu.VMEM_SHARED((_NPAD, 128), jnp.float32),
            pltpu.VMEM((_N,), jnp.float32),
            pltpu.VMEM((_N,), jnp.float32),
            pltpu.VMEM((_CH,), jnp.int32),
            pltpu.VMEM((_CH,), jnp.int32),
            pltpu.VMEM((_CH, 128), jnp.float32),
            pltpu.VMEM((_CH, 128), jnp.float32),
            pltpu.VMEM((_CH,), jnp.float32),
            pltpu.VMEM((16,), jnp.float32),
            pltpu.SemaphoreType.DMA,
        ],
    )
    return k(h1a, h1b, src, dst, elt, ert, mel, mer, zer)


def _norm1_body(sc_ref, b1_ref, w2_ref, al2_ref, ar2_ref,
                h2_ref, el2_ref, er2_ref, mel_ref, mer_ref):
    i = pl.program_id(0)
    cols = []
    for ph in range(2):
        for cc in range(2):
            plane = sc_ref[ph, cc]
            d = jnp.maximum(plane[:, 64:65], 1e-9)
            cols.append(plane[:, 0:64] / d)
    h1f = jnp.concatenate(cols, axis=1) + b1_ref[...]
    h2 = lax.dot_general(h1f, w2_ref[...], (((1,), (1,)), ((), ())),
                         preferred_element_type=jnp.float32)
    h2_ref[...] = jnp.concatenate(
        [h2, jnp.zeros((h2.shape[0], 64), jnp.float32)], axis=1)
    el2 = jnp.sum(h2 * al2_ref[0, :][None, :], axis=1)
    er2 = jnp.sum(h2 * ar2_ref[0, :][None, :], axis=1)
    el2_ref[...] = el2[:, None]
    er2_ref[...] = er2[:, None]

    @pl.when(i == 0)
    def _():
        mel_ref[...] = jnp.full((1, 16), -jnp.inf, jnp.float32)
        mer_ref[...] = jnp.full((1, 16), -jnp.inf, jnp.float32)

    mel_ref[...] = jnp.maximum(mel_ref[...], jnp.max(el2))
    mer_ref[...] = jnp.maximum(mer_ref[...], jnp.max(er2))


def _norm1_proj2(sc1, b1, W2, al2, ar2):
    return pl.pallas_call(
        _norm1_body,
        grid=(_GRID,),
        in_specs=[
            pl.BlockSpec((2, 2, _R, 128), lambda i: (0, 0, i, 0)),
            pl.BlockSpec((1, _HEADS * _HID), lambda i: (0, 0)),
            pl.BlockSpec((_HID, _HEADS * _HID), lambda i: (0, 0)),
            pl.BlockSpec((1, _HID), lambda i: (0, 0)),
            pl.BlockSpec((1, _HID), lambda i: (0, 0)),
        ],
        out_specs=[
            pl.BlockSpec((_R, 128), lambda i: (i, 0)),
            pl.BlockSpec((_R, 1), lambda i: (i, 0)),
            pl.BlockSpec((_R, 1), lambda i: (i, 0)),
            pl.BlockSpec((1, 16), lambda i: (0, 0)),
            pl.BlockSpec((1, 16), lambda i: (0, 0)),
        ],
        out_shape=[
            jax.ShapeDtypeStruct((_N, 128), jnp.float32),
            jax.ShapeDtypeStruct((_N, 1), jnp.float32),
            jax.ShapeDtypeStruct((_N, 1), jnp.float32),
            jax.ShapeDtypeStruct((1, 16), jnp.float32),
            jax.ShapeDtypeStruct((1, 16), jnp.float32),
        ],
    )(sc1, b1, W2, al2, ar2)


def _edge2_body(h2_hbm, src_hbm, dst_hbm, el_hbm, er_hbm, mel_hbm, mer_hbm, zer_hbm,
                out_hbm,
                outsh, elv, erv, srcv, dstv, featv, stg, wbuf, cbuf, sem):
    c = lax.axis_index("c")
    s = lax.axis_index("s")
    lane = lax.iota(jnp.int32, 16)
    zero16 = jnp.zeros((16,), jnp.float32)

    pltpu.sync_copy(zer_hbm, outsh.at[pl.ds(s * _ROWS_PER_SUB, _ROWS_PER_SUB), :])
    pltpu.sync_copy(el_hbm, elv)
    pltpu.sync_copy(er_hbm, erv)
    pltpu.sync_copy(mel_hbm, cbuf)
    a = cbuf[...]
    pltpu.sync_copy(mer_hbm, cbuf)
    c0 = _leaky(a + cbuf[...])

    @pl.loop(0, _CH)
    def _zs(r):
        for k in range(3):
            stg[r, pl.ds(80 + k * 16, 16)] = zero16

    plsc.subcore_barrier()

    half = _NCHUNKS // 2

    @pl.loop(0, half // _NSUB)
    def _chunk(j):
        off = (c * half + s + _NSUB * j) * _CH
        pltpu.sync_copy(src_hbm.at[pl.ds(off, _CH)], srcv)
        pltpu.sync_copy(dst_hbm.at[pl.ds(off, _CH)], dstv)
        gat = pltpu.make_async_copy(h2_hbm.at[srcv], featv, sem)
        gat.start()

        @pl.loop(0, _CH // 16)
        def _grp(g):
            src16 = srcv[pl.ds(g * 16, 16)]
            dst16 = dstv[pl.ds(g * 16, 16)]
            e0 = _leaky(plsc.load_gather(elv, [src16]) +
                        plsc.load_gather(erv, [dst16]))
            wbuf[pl.ds(g * 16, 16)] = jnp.exp(e0 - c0)

        gat.wait()

        @pl.loop(0, _CH // 16)
        def _grp2(g):
            for t in range(16):
                row = g * 16 + t
                wb = plsc.load_gather(wbuf, [jnp.full((16,), 0, jnp.int32) + row])
                for k in range(4):
                    v = featv[row, pl.ds(k * 16, 16)]
                    stg[row, pl.ds(k * 16, 16)] = v * wb
                stg[row, pl.ds(64, 16)] = jnp.where(lane == 0, wb, zero16)

        pltpu.sync_copy(stg, outsh.at[dstv], add=True)

    plsc.subcore_barrier()
    pltpu.sync_copy(outsh.at[pl.ds(s * _ROWS_PER_SUB, _ROWS_PER_SUB), :],
                    out_hbm.at[c, pl.ds(s * _ROWS_PER_SUB, _ROWS_PER_SUB), :])


def _edge_pass2(h2, src, dst, el2, er2, mel, mer):
    zer = jnp.zeros((_ROWS_PER_SUB, 128), jnp.float32)
    k = pl.kernel(
        _edge2_body,
        out_type=jax.ShapeDtypeStruct((2, _NPAD, 128), jnp.float32),
        mesh=_mesh,
        compiler_params=_SC_PARAMS,
        scratch_types=[
            pltpu.VMEM_SHARED((_NPAD, 128), jnp.float32),
            pltpu.VMEM((_N,), jnp.float32),
            pltpu.VMEM((_N,), jnp.float32),
            pltpu.VMEM((_CH,), jnp.int32),
            pltpu.VMEM((_CH,), jnp.int32),
            pltpu.VMEM((_CH, 128), jnp.float32),
            pltpu.VMEM((_CH, 128), jnp.float32),
            pltpu.VMEM((_CH,), jnp.float32),
            pltpu.VMEM((16,), jnp.float32),
            pltpu.SemaphoreType.DMA,
        ],
    )
    return k(h2, src, dst, el2, er2, mel, mer, zer)


def _final_body(p_ref, b2_ref, wn1_ref, bn1_ref, wn2_ref, bn2_ref,
                wg1_ref, bg1_ref, wg2_ref, bg2_ref,
                nl_ref, gl_ref, hsum_ref):
    i = pl.program_id(0)
    f = p_ref[0, :, 0:64] + p_ref[1, :, 0:64]
    d = jnp.maximum(p_ref[0, :, 64:65] + p_ref[1, :, 64:65], 1e-9)
    h = f / d + b2_ref[...]
    t = jnp.maximum(
        lax.dot_general(h, wn1_ref[...], (((1,), (1,)), ((), ())),
                        preferred_element_type=jnp.float32) + bn1_ref[...], 0.0)
    nl_ref[...] = lax.dot_general(t, wn2_ref[...], (((1,), (1,)), ((), ())),
                                  preferred_element_type=jnp.float32) + bn2_ref[...]

    @pl.when(i == 0)
    def _():
        hsum_ref[...] = jnp.zeros((1, _HID), jnp.float32)

    hsum_ref[...] += jnp.sum(h, axis=0, keepdims=True)

    @pl.when(i == _GRID - 1)
    def _():
        hg = hsum_ref[...] / float(_N)
        tg = jnp.maximum(
            lax.dot_general(hg, wg1_ref[...], (((1,), (1,)), ((), ())),
                            preferred_element_type=jnp.float32) + bg1_ref[...], 0.0)
        gl_ref[...] = lax.dot_general(tg, wg2_ref[...], (((1,), (1,)), ((), ())),
                                      preferred_element_type=jnp.float32) + bg2_ref[...]


def _final(p2, b2, Wn1, bn1, Wn2, bn2, Wg1, bg1, Wg2, bg2):
    return pl.pallas_call(
        _final_body,
        grid=(_GRID,),
        in_specs=[
            pl.BlockSpec((2, _R, 128), lambda i: (0, i, 0)),
            pl.BlockSpec((1, _HID), lambda i: (0, 0)),
            pl.BlockSpec((_HID, _HID), lambda i: (0, 0)),
            pl.BlockSpec((1, _HID), lambda i: (0, 0)),
            pl.BlockSpec((2, _HID), lambda i: (0, 0)),
            pl.BlockSpec((1, 2), lambda i: (0, 0)),
            pl.BlockSpec((_HID, _HID), lambda i: (0, 0)),
            pl.BlockSpec((1, _HID), lambda i: (0, 0)),
            pl.BlockSpec((2, _HID), lambda i: (0, 0)),
            pl.BlockSpec((1, 2), lambda i: (0, 0)),
        ],
        out_specs=[
            pl.BlockSpec((_R, 2), lambda i: (i, 0)),
            pl.BlockSpec((1, 2), lambda i: (0, 0)),
            pl.BlockSpec((1, _HID), lambda i: (0, 0)),
        ],
        out_shape=[
            jax.ShapeDtypeStruct((_N, 2), jnp.float32),
            jax.ShapeDtypeStruct((1, 2), jnp.float32),
            jax.ShapeDtypeStruct((1, _HID), jnp.float32),
        ],
    )(p2, b2, Wn1, bn1, Wn2, bn2, Wg1, bg1, Wg2, bg2)


def kernel(x, edge_index, W1, al1, ar1, b1, W2, al2, ar2, b2,
           Wn1, bn1, Wn2, bn2, Wg1, bg1, Wg2, bg2):
    h1a, h1b, el1, er1, mel1, mer1 = _proj1(x, W1, al1, ar1)
    src = edge_index[0]
    dst = edge_index[1]
    sc1 = _edge_pass1(h1a, h1b, src, dst, el1.T.reshape(-1), er1.T.reshape(-1),
                      mel1.reshape(-1), mer1.reshape(-1))
    h2, el2, er2, mel2, mer2 = _norm1_proj2(sc1, b1.reshape(1, -1),
                                            W2, al2, ar2)
    sc2 = _edge_pass2(h2, src, dst, el2.reshape(_N), er2.reshape(_N),
                      mel2.reshape(-1), mer2.reshape(-1))
    node_logits, graph_logits, _ = _final(sc2, b2.reshape(1, -1),
                                          Wn1, bn1.reshape(1, -1),
                                          Wn2, bn2.reshape(1, -1),
                                          Wg1, bg1.reshape(1, -1),
                                          Wg2, bg2.reshape(1, -1))
    return node_logits, graph_logits
